# baseline (speedup 1.0000x reference)
import functools

import numpy as np
import jax
import jax.numpy as jnp
from jax import lax
from jax.experimental import pallas as pl
from jax.experimental.pallas import tpu as pltpu
from jax.experimental.pallas import tpu_sc as plsc

EMBED = 32
BATCH = 16384
SEQ = 50
VOCAB = 1000000

NC = 2
NS = 16
NW = NC * NS

ROWS_PER_W = BATCH // NW
CHUNK = 64
N_CHUNKS = ROWS_PER_W // CHUNK
GROUP = 16
N_GROUPS = CHUNK // GROUP
INV_S = 1.0 / SEQ

_mesh = plsc.VectorSubcoreMesh(core_axis_name="c", subcore_axis_name="s")


@functools.partial(
    pl.kernel,
    mesh=_mesh,
    out_type=jax.ShapeDtypeStruct((BATCH, EMBED), jnp.float32),
    compiler_params=pltpu.CompilerParams(
        use_tc_tiling_on_sc=False, needs_layout_passes=False
    ),
    scratch_types=[
        pltpu.VMEM((CHUNK, SEQ), jnp.int32),
        pltpu.VMEM((CHUNK * SEQ, 16), jnp.float32),
        pltpu.VMEM((CHUNK, EMBED), jnp.float32),
        pltpu.SemaphoreType.DMA,
    ],
)
def _emb(idx_hbm, table_hbm, out_hbm, idx_v, rows_v, out_v, sem):
    wid = lax.axis_index("s") * NC + lax.axis_index("c")

    def fire(j):
        return pltpu.async_copy(
            table_hbm.at[idx_v.at[j]],
            rows_v.at[pl.ds(j * SEQ, SEQ)],
            sem,
        )

    def load_row(r):
        x = plsc.bitcast(rows_v[r, pl.ds(0, 16)], jnp.bfloat16)
        return plsc.unpack(x, format=plsc.PackFormat.INTERLEAVED)

    def chunk_body(k, carry):
        chunk_id = wid * N_CHUNKS + k
        row0 = chunk_id * CHUNK
        pltpu.sync_copy(idx_hbm.at[pl.ds(row0, CHUNK)], idx_v)
        pending = [fire(j) for j in range(GROUP)]
        for g in range(1, N_GROUPS):
            nxt = [fire(g * GROUP + j) for j in range(GROUP)]
            for c in pending:
                c.wait()
            pending = nxt
        for c in pending:
            c.wait()

        def row_body(c, carry2):
            base = c * SEQ
            a0, b0 = load_row(base)
            a1, b1 = load_row(base + 1)
            for s in range(2, SEQ, 2):
                xa, xb = load_row(base + s)
                a0 = a0 + xa
                b0 = b0 + xb
                ya, yb = load_row(base + s + 1)
                a1 = a1 + ya
                b1 = b1 + yb
            out_v[c, pl.ds(0, 16)] = (a0 + a1) * INV_S
            out_v[c, pl.ds(16, 16)] = (b0 + b1) * INV_S
            return carry2

        lax.fori_loop(0, CHUNK, row_body, 0)
        pltpu.sync_copy(out_v, out_hbm.at[pl.ds(row0, CHUNK)])
        return carry

    lax.fori_loop(0, N_CHUNKS, chunk_body, 0)


_PERM = np.arange(2 * 16).reshape(2, 16).T.reshape(-1)


def kernel(input, word_vectors):
    wvb = word_vectors[:, _PERM].astype(jnp.bfloat16)
    wv128 = jnp.pad(wvb, ((0, 0), (0, 96)))
    table4 = lax.bitcast_convert_type(
        wv128.reshape(VOCAB, 64, 2), jnp.float32
    ).reshape(4 * VOCAB, 16)
    return _emb(input.astype(jnp.int32) * 4, table4)


# speedup vs baseline: 5.5576x; 5.5576x over previous
import functools

import jax
import jax.numpy as jnp
from jax import lax
from jax.experimental import pallas as pl
from jax.experimental.pallas import tpu as pltpu
from jax.experimental.pallas import tpu_sc as plsc

EMBED = 32
BATCH = 16384
SEQ = 50
VOCAB = 1000000

NC = 2
NS = 16
NW = NC * NS

ROWS_PER_W = BATCH // NW
CHUNK = 64
N_CHUNKS = ROWS_PER_W // CHUNK
GROUP = 16
N_GROUPS = CHUNK // GROUP
INV_S = 1.0 / SEQ

_mesh = plsc.VectorSubcoreMesh(core_axis_name="c", subcore_axis_name="s")


@functools.partial(
    pl.kernel,
    mesh=_mesh,
    out_type=jax.ShapeDtypeStruct((BATCH, EMBED), jnp.float32),
    compiler_params=pltpu.CompilerParams(use_tc_tiling_on_sc=False),
    scratch_types=[
        pltpu.VMEM((CHUNK, SEQ), jnp.int32),
        pltpu.VMEM((CHUNK * SEQ, EMBED), jnp.float32),
        pltpu.VMEM((CHUNK, EMBED), jnp.float32),
        pltpu.SemaphoreType.DMA,
    ],
)
def _emb(idx_hbm, table_hbm, out_hbm, idx_v, rows_v, out_v, sem):
    wid = lax.axis_index("s") * NC + lax.axis_index("c")

    def fire(j):
        return pltpu.async_copy(
            table_hbm.at[idx_v.at[j]],
            rows_v.at[pl.ds(j * SEQ, SEQ)],
            sem,
        )

    def chunk_body(k, carry):
        chunk_id = wid * N_CHUNKS + k
        row0 = chunk_id * CHUNK
        pltpu.sync_copy(idx_hbm.at[pl.ds(row0, CHUNK)], idx_v)
        pending = [fire(j) for j in range(GROUP)]
        for g in range(1, N_GROUPS):
            nxt = [fire(g * GROUP + j) for j in range(GROUP)]
            for c in pending:
                c.wait()
            pending = nxt
        for c in pending:
            c.wait()

        def row_body(c, carry2):
            base = c * SEQ
            a0 = rows_v[base, pl.ds(0, 16)]
            a1 = rows_v[base, pl.ds(16, 16)]
            b0 = rows_v[base + 1, pl.ds(0, 16)]
            b1 = rows_v[base + 1, pl.ds(16, 16)]
            for s in range(2, SEQ, 2):
                a0 = a0 + rows_v[base + s, pl.ds(0, 16)]
                a1 = a1 + rows_v[base + s, pl.ds(16, 16)]
                b0 = b0 + rows_v[base + s + 1, pl.ds(0, 16)]
                b1 = b1 + rows_v[base + s + 1, pl.ds(16, 16)]
            out_v[c, pl.ds(0, 16)] = (a0 + b0) * INV_S
            out_v[c, pl.ds(16, 16)] = (a1 + b1) * INV_S
            return carry2

        lax.fori_loop(0, CHUNK, row_body, 0)
        pltpu.sync_copy(out_v, out_hbm.at[pl.ds(row0, CHUNK)])
        return carry

    lax.fori_loop(0, N_CHUNKS, chunk_body, 0)


def kernel(input, word_vectors):
    wv128 = jnp.pad(word_vectors, ((0, 0), (0, 3 * EMBED)))
    table4 = wv128.reshape(4 * VOCAB, EMBED)
    return _emb(input.astype(jnp.int32) * 4, table4)


# speedup vs baseline: 5.7146x; 1.0282x over previous
import functools

import jax
import jax.numpy as jnp
from jax import lax
from jax.experimental import pallas as pl
from jax.experimental.pallas import tpu as pltpu
from jax.experimental.pallas import tpu_sc as plsc

EMBED = 32
BATCH = 16384
SEQ = 50
VOCAB = 1000000

NC = 2
NS = 16
NW = NC * NS

ROWS_PER_W = BATCH // NW
CHUNK = 32
N_CHUNKS = ROWS_PER_W // CHUNK
INV_S = 1.0 / SEQ

_mesh = plsc.VectorSubcoreMesh(core_axis_name="c", subcore_axis_name="s")


@functools.partial(
    pl.kernel,
    mesh=_mesh,
    out_type=jax.ShapeDtypeStruct((BATCH, EMBED), jnp.float32),
    compiler_params=pltpu.CompilerParams(use_tc_tiling_on_sc=False),
    scratch_types=[
        pltpu.VMEM((CHUNK, SEQ), jnp.int32),
        pltpu.VMEM((CHUNK, SEQ), jnp.int32),
        pltpu.VMEM((CHUNK * SEQ, EMBED), jnp.float32),
        pltpu.VMEM((CHUNK * SEQ, EMBED), jnp.float32),
        pltpu.VMEM((CHUNK, EMBED), jnp.float32),
        pltpu.SemaphoreType.DMA,
        pltpu.SemaphoreType.DMA,
    ],
)
def _emb(idx_hbm, table_hbm, out_hbm, idx_v0, idx_v1, rows_v0, rows_v1, out_v, sem0, sem1):
    wid = lax.axis_index("s") * NC + lax.axis_index("c")
    bufs = [(idx_v0, rows_v0, sem0), (idx_v1, rows_v1, sem1)]

    def stage_and_fire(k):
        idx_v, rows_v, sem = bufs[k % 2]
        row0 = (wid * N_CHUNKS + k) * CHUNK
        pltpu.sync_copy(idx_hbm.at[pl.ds(row0, CHUNK)], idx_v)
        return [
            pltpu.async_copy(
                table_hbm.at[idx_v.at[j]],
                rows_v.at[pl.ds(j * SEQ, SEQ)],
                sem,
            )
            for j in range(CHUNK)
        ]

    def accumulate(k):
        _, rows_v, _ = bufs[k % 2]
        row0 = (wid * N_CHUNKS + k) * CHUNK

        def row_body(c, carry2):
            base = c * SEQ
            a0 = rows_v[base, pl.ds(0, 16)]
            a1 = rows_v[base, pl.ds(16, 16)]
            b0 = rows_v[base + 1, pl.ds(0, 16)]
            b1 = rows_v[base + 1, pl.ds(16, 16)]
            for s in range(2, SEQ, 2):
                a0 = a0 + rows_v[base + s, pl.ds(0, 16)]
                a1 = a1 + rows_v[base + s, pl.ds(16, 16)]
                b0 = b0 + rows_v[base + s + 1, pl.ds(0, 16)]
                b1 = b1 + rows_v[base + s + 1, pl.ds(16, 16)]
            out_v[c, pl.ds(0, 16)] = (a0 + b0) * INV_S
            out_v[c, pl.ds(16, 16)] = (a1 + b1) * INV_S
            return carry2

        lax.fori_loop(0, CHUNK, row_body, 0)
        pltpu.sync_copy(out_v, out_hbm.at[pl.ds(row0, CHUNK)])

    pending = stage_and_fire(0)
    for k in range(N_CHUNKS):
        nxt = stage_and_fire(k + 1) if k + 1 < N_CHUNKS else []
        for c in pending:
            c.wait()
        accumulate(k)
        pending = nxt


def kernel(input, word_vectors):
    wv128 = jnp.pad(word_vectors, ((0, 0), (0, 3 * EMBED)))
    table4 = wv128.reshape(4 * VOCAB, EMBED)
    return _emb(input.astype(jnp.int32) * 4, table4)
